# baseline (speedup 1.0000x reference)
import functools

import jax
import jax.numpy as jnp
from jax import lax
from jax.experimental import pallas as pl
from jax.experimental.pallas import tpu as pltpu
from jax.experimental.pallas import tpu_sc as plsc

N = 10000
E = 320000
D = 128
H = 128

NC = 2
NS = 16
CH = 128
TCH = 160
GB = 16
NG = TCH // GB
NCHUNKS = NS * TCH
EP = NCHUNKS * CH
NPAD = 10240
RPT = NPAD // NS
ZR = 128
CPT = NPAD // NS

_MM_BLK = 1000


def _mm3_body(x_ref, w1_ref, w2_ref, wr_ref, br_ref, h1_ref, h2_ref, xr_ref):
    xb = x_ref[...]
    dn = (((1,), (1,)), ((), ()))
    h1_ref[...] = lax.dot_general(xb, w1_ref[...], dn,
                                  preferred_element_type=jnp.float32)
    h2_ref[...] = lax.dot_general(xb, w2_ref[...], dn,
                                  preferred_element_type=jnp.float32)
    xr_ref[...] = lax.dot_general(xb, wr_ref[...], dn,
                                  preferred_element_type=jnp.float32) + br_ref[...]


def _combine_body(xr_ref, s1_ref, c1_ref, s2_ref, c2_ref, o_ref):
    c1 = jnp.maximum(c1_ref[...], 1.0)
    c2 = jnp.maximum(c2_ref[...], 1.0)
    o_ref[...] = jnp.maximum(
        xr_ref[...] + s1_ref[...] / c1 + s2_ref[...] / c2, 0.0)


_sc_mesh = plsc.VectorSubcoreMesh(
    core_axis_name="c", subcore_axis_name="s", num_cores=NC, num_subcores=NS)


@functools.partial(
    pl.kernel,
    out_type=(
        jax.ShapeDtypeStruct((NPAD, D), jnp.float32),
        jax.ShapeDtypeStruct((NPAD,), jnp.float32),
        jax.ShapeDtypeStruct((NPAD, D), jnp.float32),
        jax.ShapeDtypeStruct((NPAD,), jnp.float32),
    ),
    mesh=_sc_mesh,
    scratch_types=[
        pltpu.VMEM((GB, CH), jnp.int32),
        pltpu.VMEM((GB, CH), jnp.int32),
        pltpu.VMEM((GB, CH), jnp.int32),
        pltpu.VMEM((GB, CH), jnp.int32),
        pltpu.VMEM((CH, D), jnp.float32),
        pltpu.VMEM((CH, D), jnp.float32),
        pltpu.VMEM((CH,), jnp.float32),
        pltpu.VMEM_SHARED((NPAD, D), jnp.float32),
        pltpu.VMEM_SHARED((NPAD,), jnp.float32),
        pltpu.SemaphoreType.DMA,
        pltpu.SemaphoreType.DMA,
        pltpu.SemaphoreType.DMA,
        pltpu.SemaphoreType.DMA,
        pltpu.SemaphoreType.DMA,
        pltpu.SemaphoreType.DMA,
        pltpu.SemaphoreType.DMA,
        pltpu.SemaphoreType.DMA,
        pltpu.SemaphoreType.DMA,
    ],
)
def _sc_segsum(h1, h2, src_g, dst_s, dst_g, src_s, zrow_h, zcnt_h,
               sum1, cnt1, sum2, cnt2,
               gidxA, sidxA, gidxB, sidxB, rows0, rows1, ones_v,
               acc_sh, cnt_sh, sem0, sem1, semiA, semiB, sems0, sems1, semc,
               sem0b, sem1b):
    s = lax.axis_index("s")
    c = lax.axis_index("c")

    ov = jnp.ones((16,), jnp.float32)

    def fill_ones(k, _):
        ones_v[pl.ds(k * 16, 16)] = ov
        return 0
    lax.fori_loop(0, CH // 16, fill_ones, 0)

    def run_direction(h_hbm, g2d, s2d, sum_out, cnt_out):
        idx_bufs = ((gidxA, sidxA, semiA), (gidxB, sidxB, semiB))
        bufs = ((rows0, sem0, sem0b, sems0), (rows1, sem1, sem1b, sems1))

        def stage(gq, p):
            gI, sI, smi = idx_bufs[p]
            r0 = s * TCH + gq * GB
            pltpu.async_copy(g2d.at[pl.ds(r0, GB)], gI, smi)
            pltpu.async_copy(s2d.at[pl.ds(r0, GB)], sI, smi)

        def wait_stage(gq, p):
            gI, sI, smi = idx_bufs[p]
            r0 = s * TCH + gq * GB
            pltpu.make_async_copy(g2d.at[pl.ds(r0, GB)], gI, smi).wait()
            pltpu.make_async_copy(s2d.at[pl.ds(r0, GB)], sI, smi).wait()

        stage(0, 0)
        for t in range(RPT // ZR):
            pltpu.sync_copy(zrow_h, acc_sh.at[pl.ds(s * RPT + t * ZR, ZR)])
        pltpu.sync_copy(zcnt_h.at[pl.ds(s * CPT, CPT)],
                        cnt_sh.at[pl.ds(s * CPT, CPT)])
        wait_stage(0, 0)

        def fire_gather(gI, i, rb, sg, sgb):
            pltpu.async_copy(h_hbm.at[gI.at[i]], rb, sg)

        def wait_gather(gI, i, rb, sg, sgb):
            pltpu.make_async_copy(h_hbm.at[gI.at[i]], rb, sg).wait()

        fire_gather(gidxA, 0, rows0, sem0, sem0b)
        plsc.subcore_barrier()

        def outer(t, _):
            for p in range(2):
                g = 2 * t + p
                gI, sI, _ = idx_bufs[p]
                gN = idx_bufs[1 - p][0]

                @pl.when(g > 0)
                def _():
                    wait_stage(g, p)

                def pair(u, _):
                    for b in range(2):
                        i = 2 * u + b
                        rb, sg, sgb, ss = bufs[b]
                        ro, so, sob, os_ = bufs[1 - b]
                        wait_gather(gI, i, rb, sg, sgb)
                        pltpu.async_copy(rb, acc_sh.at[sI.at[i]], ss, add=True)
                        pltpu.async_copy(ones_v, cnt_sh.at[sI.at[i]], semc,
                                         add=True)
                        if b == 0:
                            @pl.when((u == 1) & (g + 1 < NG))
                            def _():
                                stage(g + 1, 1 - p)

                            @pl.when((g > 0) | (u > 0))
                            def _():
                                pltpu.make_async_copy(
                                    ro, acc_sh.at[sI.at[i]], os_).wait()
                            fire_gather(gI, i + 1, ro, so, sob)
                        else:
                            pltpu.make_async_copy(
                                ro, acc_sh.at[sI.at[i]], os_).wait()

                            @pl.when(u < GB // 2 - 1)
                            def _():
                                fire_gather(gI, i + 1, ro, so, sob)

                            @pl.when((u == GB // 2 - 1) & (g + 1 < NG))
                            def _():
                                fire_gather(gN, 0, ro, so, sob)
                    return 0
                lax.fori_loop(0, GB // 2, pair, 0)

                def cdrain(k, _):
                    pltpu.make_async_copy(
                        ones_v, cnt_sh.at[sI.at[k]], semc).wait()
                    return 0
                lax.fori_loop(0, GB, cdrain, 0)
            return 0
        lax.fori_loop(0, NG // 2, outer, 0)

        pltpu.make_async_copy(rows1, acc_sh.at[sidxB.at[0]], sems1).wait()

        plsc.subcore_barrier()
        for t in range(RPT // ZR):
            r0 = s * RPT + t * ZR
            pltpu.sync_copy(acc_sh.at[pl.ds(r0, ZR)], sum_out.at[pl.ds(r0, ZR)])
        pltpu.sync_copy(cnt_sh.at[pl.ds(s * CPT, CPT)],
                        cnt_out.at[pl.ds(s * CPT, CPT)])

    @pl.when(c == 0)
    def _():
        run_direction(h1, src_g, dst_s, sum1, cnt1)

    @pl.when(c == 1)
    def _():
        run_direction(h2, dst_g, src_s, sum2, cnt2)


def kernel(x, edge_index, W1, W2, Wr, br):
    src = edge_index[0]
    dst = edge_index[1]
    pad_g = jnp.zeros((EP - E,), jnp.int32)
    pad_s = jnp.full((EP - E,), NPAD - 1, jnp.int32)
    src_g = jnp.concatenate([src, pad_g]).reshape(NCHUNKS, CH)
    dst_s = jnp.concatenate([dst, pad_s]).reshape(NCHUNKS, CH)
    dst_g = jnp.concatenate([dst, pad_g]).reshape(NCHUNKS, CH)
    src_s = jnp.concatenate([src, pad_s]).reshape(NCHUNKS, CH)
    br2 = br.reshape(1, H)

    nblk = N // _MM_BLK
    h1, h2, xr = pl.pallas_call(
        _mm3_body,
        grid=(nblk,),
        in_specs=[
            pl.BlockSpec((_MM_BLK, D), lambda i: (i, 0)),
            pl.BlockSpec((H, D), lambda i: (0, 0)),
            pl.BlockSpec((H, D), lambda i: (0, 0)),
            pl.BlockSpec((H, D), lambda i: (0, 0)),
            pl.BlockSpec((1, H), lambda i: (0, 0)),
        ],
        out_specs=[
            pl.BlockSpec((_MM_BLK, H), lambda i: (i, 0)),
            pl.BlockSpec((_MM_BLK, H), lambda i: (i, 0)),
            pl.BlockSpec((_MM_BLK, H), lambda i: (i, 0)),
        ],
        out_shape=[
            jax.ShapeDtypeStruct((N, H), jnp.float32),
            jax.ShapeDtypeStruct((N, H), jnp.float32),
            jax.ShapeDtypeStruct((N, H), jnp.float32),
        ],
    )(x, W1, W2, Wr, br2)

    zrow_h = jnp.zeros((ZR, D), jnp.float32)
    zcnt_h = jnp.zeros((NPAD,), jnp.float32)
    sum1, cnt1, sum2, cnt2 = _sc_segsum(h1, h2, src_g, dst_s, dst_g, src_s,
                                        zrow_h, zcnt_h)

    c1 = cnt1[:N].reshape(N, 1)
    c2 = cnt2[:N].reshape(N, 1)
    out = pl.pallas_call(
        _combine_body,
        grid=(nblk,),
        in_specs=[
            pl.BlockSpec((_MM_BLK, H), lambda i: (i, 0)),
            pl.BlockSpec((_MM_BLK, H), lambda i: (i, 0)),
            pl.BlockSpec((_MM_BLK, 1), lambda i: (i, 0)),
            pl.BlockSpec((_MM_BLK, H), lambda i: (i, 0)),
            pl.BlockSpec((_MM_BLK, 1), lambda i: (i, 0)),
        ],
        out_specs=pl.BlockSpec((_MM_BLK, H), lambda i: (i, 0)),
        out_shape=jax.ShapeDtypeStruct((N, H), jnp.float32),
    )(xr, sum1, c1, sum2, c2)
    return out


# speedup vs baseline: 1.0099x; 1.0099x over previous
import functools

import jax
import jax.numpy as jnp
from jax import lax
from jax.experimental import pallas as pl
from jax.experimental.pallas import tpu as pltpu
from jax.experimental.pallas import tpu_sc as plsc

N = 10000
E = 320000
D = 128
H = 128

NC = 2
NS = 16
CH = 128
TCH = 160
GB = 16
NG = TCH // GB
NCHUNKS = NS * TCH
EP = NCHUNKS * CH
NPAD = 10240
RPT = NPAD // NS
ZR = 128
CPT = NPAD // NS

_MM_BLK = 2000


def _mm3_body(x_ref, w1_ref, w2_ref, wr_ref, br_ref, h1_ref, h2_ref, xr_ref):
    xb = x_ref[...]
    dn = (((1,), (1,)), ((), ()))
    h1_ref[...] = lax.dot_general(xb, w1_ref[...], dn,
                                  preferred_element_type=jnp.float32)
    h2_ref[...] = lax.dot_general(xb, w2_ref[...], dn,
                                  preferred_element_type=jnp.float32)
    xr_ref[...] = lax.dot_general(xb, wr_ref[...], dn,
                                  preferred_element_type=jnp.float32) + br_ref[...]


def _combine_body(xr_ref, s1_ref, c1_ref, s2_ref, c2_ref, o_ref):
    c1 = jnp.maximum(c1_ref[...], 1.0)
    c2 = jnp.maximum(c2_ref[...], 1.0)
    o_ref[...] = jnp.maximum(
        xr_ref[...] + s1_ref[...] / c1 + s2_ref[...] / c2, 0.0)


_sc_mesh = plsc.VectorSubcoreMesh(
    core_axis_name="c", subcore_axis_name="s", num_cores=NC, num_subcores=NS)


@functools.partial(
    pl.kernel,
    out_type=(
        jax.ShapeDtypeStruct((NPAD, D), jnp.float32),
        jax.ShapeDtypeStruct((NPAD,), jnp.float32),
        jax.ShapeDtypeStruct((NPAD, D), jnp.float32),
        jax.ShapeDtypeStruct((NPAD,), jnp.float32),
    ),
    mesh=_sc_mesh,
    scratch_types=[
        pltpu.VMEM((GB, CH), jnp.int32),
        pltpu.VMEM((GB, CH), jnp.int32),
        pltpu.VMEM((GB, CH), jnp.int32),
        pltpu.VMEM((GB, CH), jnp.int32),
        pltpu.VMEM((CH, D), jnp.float32),
        pltpu.VMEM((CH, D), jnp.float32),
        pltpu.VMEM((CH,), jnp.float32),
        pltpu.VMEM_SHARED((NPAD, D), jnp.float32),
        pltpu.VMEM_SHARED((NPAD,), jnp.float32),
        pltpu.SemaphoreType.DMA,
        pltpu.SemaphoreType.DMA,
        pltpu.SemaphoreType.DMA,
        pltpu.SemaphoreType.DMA,
        pltpu.SemaphoreType.DMA,
        pltpu.SemaphoreType.DMA,
        pltpu.SemaphoreType.DMA,
        pltpu.SemaphoreType.DMA,
        pltpu.SemaphoreType.DMA,
    ],
)
def _sc_segsum(h1, h2, src_g, dst_s, dst_g, src_s, zrow_h, zcnt_h,
               sum1, cnt1, sum2, cnt2,
               gidxA, sidxA, gidxB, sidxB, rows0, rows1, ones_v,
               acc_sh, cnt_sh, sem0, sem1, semiA, semiB, sems0, sems1, semc,
               sem0b, sem1b):
    s = lax.axis_index("s")
    c = lax.axis_index("c")

    ov = jnp.ones((16,), jnp.float32)

    def fill_ones(k, _):
        ones_v[pl.ds(k * 16, 16)] = ov
        return 0
    lax.fori_loop(0, CH // 16, fill_ones, 0)

    def run_direction(h_hbm, g2d, s2d, sum_out, cnt_out):
        idx_bufs = ((gidxA, sidxA, semiA), (gidxB, sidxB, semiB))
        bufs = ((rows0, sem0, sem0b, sems0), (rows1, sem1, sem1b, sems1))

        def stage(gq, p):
            gI, sI, smi = idx_bufs[p]
            r0 = s * TCH + gq * GB
            pltpu.async_copy(g2d.at[pl.ds(r0, GB)], gI, smi)
            pltpu.async_copy(s2d.at[pl.ds(r0, GB)], sI, smi)

        def wait_stage(gq, p):
            gI, sI, smi = idx_bufs[p]
            r0 = s * TCH + gq * GB
            pltpu.make_async_copy(g2d.at[pl.ds(r0, GB)], gI, smi).wait()
            pltpu.make_async_copy(s2d.at[pl.ds(r0, GB)], sI, smi).wait()

        stage(0, 0)
        for t in range(RPT // ZR):
            pltpu.sync_copy(zrow_h, acc_sh.at[pl.ds(s * RPT + t * ZR, ZR)])
        pltpu.sync_copy(zcnt_h.at[pl.ds(s * CPT, CPT)],
                        cnt_sh.at[pl.ds(s * CPT, CPT)])
        wait_stage(0, 0)

        def fire_gather(gI, i, rb, sg, sgb):
            pltpu.async_copy(h_hbm.at[gI.at[i]], rb, sg)

        def wait_gather(gI, i, rb, sg, sgb):
            pltpu.make_async_copy(h_hbm.at[gI.at[i]], rb, sg).wait()

        fire_gather(gidxA, 0, rows0, sem0, sem0b)
        plsc.subcore_barrier()

        def outer(t, _):
            for p in range(2):
                g = 2 * t + p
                gI, sI, _ = idx_bufs[p]
                gN = idx_bufs[1 - p][0]

                @pl.when(g > 0)
                def _():
                    wait_stage(g, p)

                def pair(u, _):
                    for b in range(2):
                        i = 2 * u + b
                        rb, sg, sgb, ss = bufs[b]
                        ro, so, sob, os_ = bufs[1 - b]
                        wait_gather(gI, i, rb, sg, sgb)
                        pltpu.async_copy(rb, acc_sh.at[sI.at[i]], ss, add=True)
                        pltpu.async_copy(ones_v, cnt_sh.at[sI.at[i]], semc,
                                         add=True)
                        if b == 0:
                            @pl.when((u == 1) & (g + 1 < NG))
                            def _():
                                stage(g + 1, 1 - p)

                            @pl.when((g > 0) | (u > 0))
                            def _():
                                pltpu.make_async_copy(
                                    ro, acc_sh.at[sI.at[i]], os_).wait()
                            fire_gather(gI, i + 1, ro, so, sob)
                        else:
                            pltpu.make_async_copy(
                                ro, acc_sh.at[sI.at[i]], os_).wait()

                            @pl.when(u < GB // 2 - 1)
                            def _():
                                fire_gather(gI, i + 1, ro, so, sob)

                            @pl.when((u == GB // 2 - 1) & (g + 1 < NG))
                            def _():
                                fire_gather(gN, 0, ro, so, sob)
                    return 0
                lax.fori_loop(0, GB // 2, pair, 0)

                def cdrain(k, _):
                    pltpu.make_async_copy(
                        ones_v, cnt_sh.at[sI.at[k]], semc).wait()
                    return 0
                lax.fori_loop(0, GB, cdrain, 0)
            return 0
        lax.fori_loop(0, NG // 2, outer, 0)

        pltpu.make_async_copy(rows1, acc_sh.at[sidxB.at[0]], sems1).wait()

        plsc.subcore_barrier()
        for t in range(RPT // ZR):
            r0 = s * RPT + t * ZR
            pltpu.sync_copy(acc_sh.at[pl.ds(r0, ZR)], sum_out.at[pl.ds(r0, ZR)])
        pltpu.sync_copy(cnt_sh.at[pl.ds(s * CPT, CPT)],
                        cnt_out.at[pl.ds(s * CPT, CPT)])

    @pl.when(c == 0)
    def _():
        run_direction(h1, src_g, dst_s, sum1, cnt1)

    @pl.when(c == 1)
    def _():
        run_direction(h2, dst_g, src_s, sum2, cnt2)


def kernel(x, edge_index, W1, W2, Wr, br):
    src = edge_index[0]
    dst = edge_index[1]
    pad_g = jnp.zeros((EP - E,), jnp.int32)
    pad_s = jnp.full((EP - E,), NPAD - 1, jnp.int32)
    src_g = jnp.concatenate([src, pad_g]).reshape(NCHUNKS, CH)
    dst_s = jnp.concatenate([dst, pad_s]).reshape(NCHUNKS, CH)
    dst_g = jnp.concatenate([dst, pad_g]).reshape(NCHUNKS, CH)
    src_s = jnp.concatenate([src, pad_s]).reshape(NCHUNKS, CH)
    br2 = br.reshape(1, H)

    nblk = N // _MM_BLK
    h1, h2, xr = pl.pallas_call(
        _mm3_body,
        grid=(nblk,),
        in_specs=[
            pl.BlockSpec((_MM_BLK, D), lambda i: (i, 0)),
            pl.BlockSpec((H, D), lambda i: (0, 0)),
            pl.BlockSpec((H, D), lambda i: (0, 0)),
            pl.BlockSpec((H, D), lambda i: (0, 0)),
            pl.BlockSpec((1, H), lambda i: (0, 0)),
        ],
        out_specs=[
            pl.BlockSpec((_MM_BLK, H), lambda i: (i, 0)),
            pl.BlockSpec((_MM_BLK, H), lambda i: (i, 0)),
            pl.BlockSpec((_MM_BLK, H), lambda i: (i, 0)),
        ],
        out_shape=[
            jax.ShapeDtypeStruct((N, H), jnp.float32),
            jax.ShapeDtypeStruct((N, H), jnp.float32),
            jax.ShapeDtypeStruct((N, H), jnp.float32),
        ],
    )(x, W1, W2, Wr, br2)

    zrow_h = jnp.zeros((ZR, D), jnp.float32)
    zcnt_h = jnp.zeros((NPAD,), jnp.float32)
    sum1, cnt1, sum2, cnt2 = _sc_segsum(h1, h2, src_g, dst_s, dst_g, src_s,
                                        zrow_h, zcnt_h)

    c1 = cnt1[:N].reshape(N, 1)
    c2 = cnt2[:N].reshape(N, 1)
    out = pl.pallas_call(
        _combine_body,
        grid=(nblk,),
        in_specs=[
            pl.BlockSpec((_MM_BLK, H), lambda i: (i, 0)),
            pl.BlockSpec((_MM_BLK, H), lambda i: (i, 0)),
            pl.BlockSpec((_MM_BLK, 1), lambda i: (i, 0)),
            pl.BlockSpec((_MM_BLK, H), lambda i: (i, 0)),
            pl.BlockSpec((_MM_BLK, 1), lambda i: (i, 0)),
        ],
        out_specs=pl.BlockSpec((_MM_BLK, H), lambda i: (i, 0)),
        out_shape=jax.ShapeDtypeStruct((N, H), jnp.float32),
    )(xr, sum1, c1, sum2, c2)
    return out


# speedup vs baseline: 1.0107x; 1.0008x over previous
import functools

import jax
import jax.numpy as jnp
from jax import lax
from jax.experimental import pallas as pl
from jax.experimental.pallas import tpu as pltpu
from jax.experimental.pallas import tpu_sc as plsc

N = 10000
E = 320000
D = 128
H = 128

NC = 2
NS = 16
CH = 128
TCH = 160
GB = 16
NG = TCH // GB
NCHUNKS = NS * TCH
EP = NCHUNKS * CH
NPAD = 10240
RPT = NPAD // NS
ZR = 128
CPT = NPAD // NS

_MM_BLK = 2000


def _mm3_body(x_ref, w1_ref, w2_ref, wr_ref, br_ref, h1_ref, h2_ref, xr_ref):
    xb = x_ref[...]
    dn = (((1,), (1,)), ((), ()))
    h1_ref[...] = lax.dot_general(xb, w1_ref[...], dn,
                                  preferred_element_type=jnp.float32)
    h2_ref[...] = lax.dot_general(xb, w2_ref[...], dn,
                                  preferred_element_type=jnp.float32)
    xr_ref[...] = lax.dot_general(xb, wr_ref[...], dn,
                                  preferred_element_type=jnp.float32) + br_ref[...]


def _combine_body(xr_ref, s1_ref, c1_ref, s2_ref, c2_ref, o_ref):
    c1 = jnp.maximum(c1_ref[...], 1.0)
    c2 = jnp.maximum(c2_ref[...], 1.0)
    o_ref[...] = jnp.maximum(
        xr_ref[...] + s1_ref[...] / c1 + s2_ref[...] / c2, 0.0)


_sc_mesh = plsc.VectorSubcoreMesh(
    core_axis_name="c", subcore_axis_name="s", num_cores=NC, num_subcores=NS)


@functools.partial(
    pl.kernel,
    out_type=(
        jax.ShapeDtypeStruct((NPAD, D), jnp.float32),
        jax.ShapeDtypeStruct((NPAD,), jnp.float32),
        jax.ShapeDtypeStruct((NPAD, D), jnp.float32),
        jax.ShapeDtypeStruct((NPAD,), jnp.float32),
    ),
    mesh=_sc_mesh,
    scratch_types=[
        pltpu.VMEM((GB, CH), jnp.int32),
        pltpu.VMEM((GB, CH), jnp.int32),
        pltpu.VMEM((GB, CH), jnp.int32),
        pltpu.VMEM((GB, CH), jnp.int32),
        pltpu.VMEM((CH, D), jnp.float32),
        pltpu.VMEM((CH, D), jnp.float32),
        pltpu.VMEM((CH,), jnp.float32),
        pltpu.VMEM_SHARED((NPAD, D), jnp.float32),
        pltpu.VMEM_SHARED((NPAD,), jnp.float32),
        pltpu.SemaphoreType.DMA,
        pltpu.SemaphoreType.DMA,
        pltpu.SemaphoreType.DMA,
        pltpu.SemaphoreType.DMA,
        pltpu.SemaphoreType.DMA,
        pltpu.SemaphoreType.DMA,
        pltpu.SemaphoreType.DMA,
    ],
)
def _sc_segsum(h1, h2, src_g, dst_s, dst_g, src_s, zrow_h, zcnt_h,
               sum1, cnt1, sum2, cnt2,
               gidxA, sidxA, gidxB, sidxB, rows0, rows1, ones_v,
               acc_sh, cnt_sh, sem0, sem1, semiA, semiB, sems0, sems1, semc):
    s = lax.axis_index("s")
    c = lax.axis_index("c")

    ov = jnp.ones((16,), jnp.float32)

    def fill_ones(k, _):
        ones_v[pl.ds(k * 16, 16)] = ov
        return 0
    lax.fori_loop(0, CH // 16, fill_ones, 0)

    def run_direction(h_hbm, g2d, s2d, sum_out, cnt_out):
        idx_bufs = ((gidxA, sidxA, semiA), (gidxB, sidxB, semiB))
        bufs = ((rows0, sem0, sems0), (rows1, sem1, sems1))

        def stage(gq, p):
            gI, sI, smi = idx_bufs[p]
            r0 = s * TCH + gq * GB
            pltpu.async_copy(g2d.at[pl.ds(r0, GB)], gI, smi)
            pltpu.async_copy(s2d.at[pl.ds(r0, GB)], sI, smi)

        def wait_stage(gq, p):
            gI, sI, smi = idx_bufs[p]
            r0 = s * TCH + gq * GB
            pltpu.make_async_copy(g2d.at[pl.ds(r0, GB)], gI, smi).wait()
            pltpu.make_async_copy(s2d.at[pl.ds(r0, GB)], sI, smi).wait()

        stage(0, 0)
        for t in range(RPT // ZR):
            pltpu.sync_copy(zrow_h, acc_sh.at[pl.ds(s * RPT + t * ZR, ZR)])
        pltpu.sync_copy(zcnt_h.at[pl.ds(s * CPT, CPT)],
                        cnt_sh.at[pl.ds(s * CPT, CPT)])
        wait_stage(0, 0)

        def fire_gather(gI, i, rb, sg):
            pltpu.async_copy(h_hbm.at[gI.at[i]], rb, sg)

        def wait_gather(gI, i, rb, sg):
            pltpu.make_async_copy(h_hbm.at[gI.at[i]], rb, sg).wait()

        fire_gather(gidxA, 0, rows0, sem0)
        plsc.subcore_barrier()

        def outer(t, _):
            for p in range(2):
                g = 2 * t + p
                gI, sI, _ = idx_bufs[p]
                gN = idx_bufs[1 - p][0]

                @pl.when(g > 0)
                def _():
                    wait_stage(g, p)

                def pair(u, _):
                    for b in range(2):
                        i = 2 * u + b
                        rb, sg, ss = bufs[b]
                        ro, so, os_ = bufs[1 - b]
                        wait_gather(gI, i, rb, sg)
                        pltpu.async_copy(rb, acc_sh.at[sI.at[i]], ss, add=True)
                        pltpu.async_copy(ones_v, cnt_sh.at[sI.at[i]], semc,
                                         add=True)
                        if b == 0:
                            @pl.when((u == 1) & (g + 1 < NG))
                            def _():
                                stage(g + 1, 1 - p)

                            @pl.when((g > 0) | (u > 0))
                            def _():
                                pltpu.make_async_copy(
                                    ro, acc_sh.at[sI.at[i]], os_).wait()
                            fire_gather(gI, i + 1, ro, so)
                        else:
                            pltpu.make_async_copy(
                                ro, acc_sh.at[sI.at[i]], os_).wait()

                            @pl.when(u < GB // 2 - 1)
                            def _():
                                fire_gather(gI, i + 1, ro, so)

                            @pl.when((u == GB // 2 - 1) & (g + 1 < NG))
                            def _():
                                fire_gather(gN, 0, ro, so)
                    return 0
                lax.fori_loop(0, GB // 2, pair, 0)

                def cdrain(k, _):
                    pltpu.make_async_copy(
                        ones_v, cnt_sh.at[sI.at[k]], semc).wait()
                    return 0
                lax.fori_loop(0, GB, cdrain, 0)
            return 0
        lax.fori_loop(0, NG // 2, outer, 0)

        pltpu.make_async_copy(rows1, acc_sh.at[sidxB.at[0]], sems1).wait()

        plsc.subcore_barrier()
        for t in range(RPT // ZR):
            r0 = s * RPT + t * ZR
            pltpu.sync_copy(acc_sh.at[pl.ds(r0, ZR)], sum_out.at[pl.ds(r0, ZR)])
        pltpu.sync_copy(cnt_sh.at[pl.ds(s * CPT, CPT)],
                        cnt_out.at[pl.ds(s * CPT, CPT)])

    @pl.when(c == 0)
    def _():
        run_direction(h1, src_g, dst_s, sum1, cnt1)

    @pl.when(c == 1)
    def _():
        run_direction(h2, dst_g, src_s, sum2, cnt2)


def kernel(x, edge_index, W1, W2, Wr, br):
    src = edge_index[0]
    dst = edge_index[1]
    pad_g = jnp.zeros((EP - E,), jnp.int32)
    pad_s = jnp.full((EP - E,), NPAD - 1, jnp.int32)
    src_g = jnp.concatenate([src, pad_g]).reshape(NCHUNKS, CH)
    dst_s = jnp.concatenate([dst, pad_s]).reshape(NCHUNKS, CH)
    dst_g = jnp.concatenate([dst, pad_g]).reshape(NCHUNKS, CH)
    src_s = jnp.concatenate([src, pad_s]).reshape(NCHUNKS, CH)
    br2 = br.reshape(1, H)

    nblk = N // _MM_BLK
    h1, h2, xr = pl.pallas_call(
        _mm3_body,
        grid=(nblk,),
        in_specs=[
            pl.BlockSpec((_MM_BLK, D), lambda i: (i, 0)),
            pl.BlockSpec((H, D), lambda i: (0, 0)),
            pl.BlockSpec((H, D), lambda i: (0, 0)),
            pl.BlockSpec((H, D), lambda i: (0, 0)),
            pl.BlockSpec((1, H), lambda i: (0, 0)),
        ],
        out_specs=[
            pl.BlockSpec((_MM_BLK, H), lambda i: (i, 0)),
            pl.BlockSpec((_MM_BLK, H), lambda i: (i, 0)),
            pl.BlockSpec((_MM_BLK, H), lambda i: (i, 0)),
        ],
        out_shape=[
            jax.ShapeDtypeStruct((N, H), jnp.float32),
            jax.ShapeDtypeStruct((N, H), jnp.float32),
            jax.ShapeDtypeStruct((N, H), jnp.float32),
        ],
    )(x, W1, W2, Wr, br2)

    zrow_h = jnp.zeros((ZR, D), jnp.float32)
    zcnt_h = jnp.zeros((NPAD,), jnp.float32)
    sum1, cnt1, sum2, cnt2 = _sc_segsum(h1, h2, src_g, dst_s, dst_g, src_s,
                                        zrow_h, zcnt_h)

    c1 = cnt1[:N].reshape(N, 1)
    c2 = cnt2[:N].reshape(N, 1)
    out = pl.pallas_call(
        _combine_body,
        grid=(nblk,),
        in_specs=[
            pl.BlockSpec((_MM_BLK, H), lambda i: (i, 0)),
            pl.BlockSpec((_MM_BLK, H), lambda i: (i, 0)),
            pl.BlockSpec((_MM_BLK, 1), lambda i: (i, 0)),
            pl.BlockSpec((_MM_BLK, H), lambda i: (i, 0)),
            pl.BlockSpec((_MM_BLK, 1), lambda i: (i, 0)),
        ],
        out_specs=pl.BlockSpec((_MM_BLK, H), lambda i: (i, 0)),
        out_shape=jax.ShapeDtypeStruct((N, H), jnp.float32),
    )(xr, sum1, c1, sum2, c2)
    return out
